# baseline (speedup 1.0000x reference)
import functools

import jax
import jax.numpy as jnp
from jax import lax
from jax.experimental import pallas as pl
from jax.experimental.pallas import tpu as pltpu
from jax.experimental.pallas import tpu_sc as plsc

B = 8
H = 224
W = 224
C = 96
CP = 128
NC = 2
NS = 16
NW = NC * NS
ROWS_PER_TILE = H * B // NW
HALF = W // 2
SUB0 = 64
NPIX = B * H * W


def _bf16r(v):
    b = lax.bitcast_convert_type(v, jnp.int32)
    lsb = lax.shift_right_logical(b, 16) & 1
    r = (b + 32767 + lsb) & jnp.int32(-65536)
    return lax.bitcast_convert_type(r, jnp.float32)


def _stn_body(table, thetap, out, idx_a, idx_b, idx_c, idx_d,
              gb_a, gb_b, gb_c, gb_d, wbuf, obuf, tvm, flag,
              gsem0, gsem1, osem0, osem1):
    wid = lax.axis_index("s") * NC + lax.axis_index("c")
    bidx = wid // 4
    q = wid % 4
    row_base = q * ROWS_PER_TILE

    pltpu.sync_copy(thetap.at[pl.ds(bidx, 1)], tvm)
    tvec = _bf16r(tvm[0, pl.ds(0, 16)])
    t00 = tvec[0]
    t01 = tvec[1]
    t02 = tvec[2]
    t10 = tvec[3]
    t11 = tvec[4]
    t12 = tvec[5]

    zv = jnp.zeros((16,), jnp.float32)
    iota = lax.iota(jnp.int32, 16)
    iota_f = iota.astype(jnp.float32)
    step = jnp.float32(2.0 / (W - 1))
    halfw = jnp.float32(W / 2.0)
    idxs = (idx_a, idx_b, idx_c, idx_d)
    gbs = (gb_a, gb_b, gb_c, gb_d)
    gsems = (gsem0, gsem1)
    osems = (osem0, osem1)

    def chunk(r, half, s):
        i_f = (row_base + r).astype(jnp.float32)
        yt = _bf16r(i_f * step - 1.0)
        sx = t01 * yt + t02 + 1.0
        sy = t11 * yt + t12 + 1.0
        vacc = zv
        for k in range(HALF // 16):
            j_f = iota_f + jnp.float32(half * HALF + k * 16)
            xt = _bf16r(j_f * step - 1.0)
            x = (t00 * xt + sx) * halfw
            y = (t10 * xt + sy) * halfw
            xi = x.astype(jnp.int32)
            x0i = xi - (xi.astype(jnp.float32) > x).astype(jnp.int32)
            yi = y.astype(jnp.int32)
            y0i = yi - (yi.astype(jnp.float32) > y).astype(jnp.int32)
            valid = ((x >= 0.0) & (x < jnp.float32(W - 1))
                     & (y >= 0.0) & (y < jnp.float32(H - 1)))
            vf = jnp.where(valid, jnp.float32(1.0), jnp.float32(0.0))
            vacc = jnp.maximum(vacc, vf)
            x0c = jnp.clip(x0i, 0, W - 2)
            y0c = jnp.clip(y0i, 0, H - 2)
            fx = x - x0c.astype(jnp.float32)
            fy = y - y0c.astype(jnp.float32)
            gx = 1.0 - fx
            gy = 1.0 - fy
            sl = pl.ds(k * 16, 16)
            wbuf[0, sl] = gx * gy * vf
            wbuf[1, sl] = gx * fy * vf
            wbuf[2, sl] = fx * gy * vf
            wbuf[3, sl] = fx * fy * vf
            ia = bidx * (H * W) + y0c * W + x0c
            idx_a[sl] = ia
            idx_b[sl] = ia + W
            idx_c[sl] = ia + 1
            idx_d[sl] = ia + (W + 1)
        anyv = jnp.max(vacc) > 0.0
        flag[half] = anyv.astype(jnp.int32)

        @pl.when(anyv)
        def _():
            for k in range(4):
                pltpu.async_copy(table.at[idxs[k].at[pl.ds(0, SUB0)]],
                                 gbs[k].at[pl.ds(0, SUB0)], gsems[0])
            for k in range(4):
                pltpu.async_copy(
                    table.at[idxs[k].at[pl.ds(SUB0, HALF - SUB0)]],
                    gbs[k].at[pl.ds(SUB0, HALF - SUB0)], gsems[1])

            def grp(g, carry):
                slw = pl.ds(g * 16, 16)
                wav = wbuf[0, slw]
                wbv = wbuf[1, slw]
                wcv = wbuf[2, slw]
                wdv = wbuf[3, slw]
                pv = g * 16 + iota
                slo = pl.ds(half * HALF + g * 16, 16)

                def chan(cc, carry2):
                    cs = jnp.full((16,), cc, jnp.int32)
                    for c8 in range(8):
                        cvec = cs + c8
                        a = plsc.load_gather(gb_a, [pv, cvec])
                        bb = plsc.load_gather(gb_b, [pv, cvec])
                        cg = plsc.load_gather(gb_c, [pv, cvec])
                        dd = plsc.load_gather(gb_d, [pv, cvec])
                        acc = wav * a + wbv * bb + wcv * cg + wdv * dd
                        obuf[s, cc + c8, slo] = acc
                    return carry2

                lax.fori_loop(0, C // 8, lambda t, cy: chan(t * 8, cy), 0)
                return carry

            for k in range(4):
                pltpu.make_async_copy(table.at[idxs[k].at[pl.ds(0, SUB0)]],
                                      gbs[k].at[pl.ds(0, SUB0)],
                                      gsems[0]).wait()
            lax.fori_loop(0, SUB0 // 16, grp, 0)
            for k in range(4):
                pltpu.make_async_copy(
                    table.at[idxs[k].at[pl.ds(SUB0, HALF - SUB0)]],
                    gbs[k].at[pl.ds(SUB0, HALF - SUB0)], gsems[1]).wait()
            lax.fori_loop(SUB0 // 16, HALF // 16, grp, 0)

    def zfill_half(s, half):
        def zh(cc, carry):
            for v in range(HALF // 16):
                obuf[s, cc, pl.ds(half * HALF + v * 16, 16)] = zv
            return carry

        lax.fori_loop(0, C, zh, 0)

    def do_row(r, s, rr):
        row_id = bidx * H + row_base + r
        owin = out.at[row_id]

        @pl.when(rr > 0)
        def _():
            pltpu.make_async_copy(obuf.at[s], owin, osems[s]).wait()

        chunk(r, 0, s)
        chunk(r, 1, s)
        f0 = flag[0]
        f1 = flag[1]

        @pl.when(f0 == 0)
        def _():
            zfill_half(s, 0)

        @pl.when(f1 == 0)
        def _():
            zfill_half(s, 1)

        pltpu.async_copy(obuf.at[s], owin, osems[s])

    def pair_body(rr, carry):
        do_row(rr * 2, 0, rr)
        do_row(rr * 2 + 1, 1, rr)
        return carry

    lax.fori_loop(0, ROWS_PER_TILE // 2, pair_body, 0)

    row_last = bidx * H + row_base + ROWS_PER_TILE - 1
    for s in range(2):
        pltpu.make_async_copy(obuf.at[s], out.at[row_last], osems[s]).wait()


@functools.partial(
    pl.kernel,
    out_type=jax.ShapeDtypeStruct((B * H, C, W), jnp.float32),
    mesh=plsc.VectorSubcoreMesh(core_axis_name="c", subcore_axis_name="s",
                                num_cores=NC, num_subcores=NS),
    compiler_params=pltpu.CompilerParams(needs_layout_passes=False,
                                         use_tc_tiling_on_sc=True),
    scratch_types=[
        pltpu.VMEM((HALF,), jnp.int32),
        pltpu.VMEM((HALF,), jnp.int32),
        pltpu.VMEM((HALF,), jnp.int32),
        pltpu.VMEM((HALF,), jnp.int32),
        pltpu.VMEM((HALF, CP), jnp.float32),
        pltpu.VMEM((HALF, CP), jnp.float32),
        pltpu.VMEM((HALF, CP), jnp.float32),
        pltpu.VMEM((HALF, CP), jnp.float32),
        pltpu.VMEM((4, HALF), jnp.float32),
        pltpu.VMEM((2, C, W), jnp.float32),
        pltpu.VMEM((1, 16), jnp.float32),
        pltpu.SMEM((2,), jnp.int32),
        pltpu.SemaphoreType.DMA,
        pltpu.SemaphoreType.DMA,
        pltpu.SemaphoreType.DMA,
        pltpu.SemaphoreType.DMA,
    ],
)
def _stn_call(table, thetap, out, *scratch):
    _stn_body(table, thetap, out, *scratch)


def kernel(conv_input, theta):
    table = conv_input.reshape(NPIX, C)
    table = jnp.concatenate(
        [table, jnp.zeros((NPIX, CP - C), jnp.float32)], axis=1)
    thetap = jnp.zeros((B, 16), jnp.float32).at[:, :6].set(
        theta.astype(jnp.float32))
    out = _stn_call(table, thetap)
    return jnp.swapaxes(out.reshape(B, H, C, W), 2, 3)


# speedup vs baseline: 2.4939x; 2.4939x over previous
import functools

import jax
import jax.numpy as jnp
from jax import lax
from jax.experimental import pallas as pl
from jax.experimental.pallas import tpu as pltpu
from jax.experimental.pallas import tpu_sc as plsc

B = 8
H = 224
W = 224
C = 96
CP = 128
NC = 2
NS = 16
NW = NC * NS
NPIX = B * H * W
PIX_PER_TILE = NPIX // NW
CHUNK = 64
CHUNKS_PER_TILE = PIX_PER_TILE // CHUNK


def _bf16r(v):
    b = lax.bitcast_convert_type(v, jnp.int32)
    lsb = lax.shift_right_logical(b, 16) & 1
    r = (b + 32767 + lsb) & jnp.int32(-65536)
    return lax.bitcast_convert_type(r, jnp.float32)


def _stn_body(table, thetap, out, idx_a, idx_b, idx_c, idx_d,
              gb_a, gb_b, gb_c, gb_d, wbuf, obuf, zbuf, tvm, flag,
              gsem0, gsem1, osem0, osem1):
    wid = lax.axis_index("s") * NC + lax.axis_index("c")
    tile_pix_base = wid * PIX_PER_TILE
    bidx = wid // 4

    pltpu.sync_copy(thetap.at[pl.ds(bidx, 1)], tvm)
    tvec = _bf16r(tvm[0, pl.ds(0, 16)])
    t00 = tvec[0]
    t01 = tvec[1]
    t02 = tvec[2]
    t10 = tvec[3]
    t11 = tvec[4]
    t12 = tvec[5]

    zv = jnp.zeros((16,), jnp.float32)
    iota = lax.iota(jnp.int32, 16)
    step = jnp.float32(2.0 / (W - 1))
    halfw = jnp.float32(W / 2.0)
    idxs = (idx_a, idx_b, idx_c, idx_d)
    gbs = (gb_a, gb_b, gb_c, gb_d)
    gsems = (gsem0, gsem1)
    osems = (osem0, osem1)

    def zfill(p, carry):
        for v in range(C // 16):
            zbuf[p, pl.ds(v * 16, 16)] = zv
        return carry

    lax.fori_loop(0, CHUNK, zfill, 0)

    def coords_and_fire(cidx, s):
        pix0 = tile_pix_base + cidx * CHUNK
        vacc = zv
        for k in range(CHUNK // 16):
            p_vec = pix0 + (k * 16) + iota - bidx * (H * W)
            i_vec = p_vec // W
            j_vec = p_vec - i_vec * W
            yt = _bf16r(i_vec.astype(jnp.float32) * step - 1.0)
            xt = _bf16r(j_vec.astype(jnp.float32) * step - 1.0)
            x = (t00 * xt + (t01 * yt + t02 + 1.0)) * halfw
            y = (t10 * xt + (t11 * yt + t12 + 1.0)) * halfw
            xi = x.astype(jnp.int32)
            x0i = xi - (xi.astype(jnp.float32) > x).astype(jnp.int32)
            yi = y.astype(jnp.int32)
            y0i = yi - (yi.astype(jnp.float32) > y).astype(jnp.int32)
            valid = ((x >= 0.0) & (x < jnp.float32(W - 1))
                     & (y >= 0.0) & (y < jnp.float32(H - 1)))
            vf = jnp.where(valid, jnp.float32(1.0), jnp.float32(0.0))
            vacc = jnp.maximum(vacc, vf)
            x0c = jnp.clip(x0i, 0, W - 2)
            y0c = jnp.clip(y0i, 0, H - 2)
            fx = x - x0c.astype(jnp.float32)
            fy = y - y0c.astype(jnp.float32)
            gx = 1.0 - fx
            gy = 1.0 - fy
            sl = pl.ds(k * 16, 16)
            wbuf[s, 0, sl] = gx * gy * vf
            wbuf[s, 1, sl] = gx * fy * vf
            wbuf[s, 2, sl] = fx * gy * vf
            wbuf[s, 3, sl] = fx * fy * vf
            ia = bidx * (H * W) + y0c * W + x0c
            idx_a[s, sl] = ia
            idx_b[s, sl] = ia + W
            idx_c[s, sl] = ia + 1
            idx_d[s, sl] = ia + (W + 1)
        anyv = jnp.max(vacc) > 0.0
        flag[s] = anyv.astype(jnp.int32)

        @pl.when(anyv)
        def _():
            for k in range(4):
                pltpu.async_copy(table.at[idxs[k].at[s]], gbs[k].at[s],
                                 gsems[s])

    def drain_combine_out(cidx, s, first):
        pix_base = tile_pix_base + cidx * CHUNK
        fl = flag[s]

        if first is None:
            pltpu.make_async_copy(obuf.at[s], out.at[pl.ds(pix_base, CHUNK)],
                                  osems[s]).wait()
        else:
            @pl.when(~first)
            def _():
                pltpu.make_async_copy(obuf.at[s],
                                      out.at[pl.ds(pix_base, CHUNK)],
                                      osems[s]).wait()

        @pl.when(fl == 1)
        def _():
            for k in range(4):
                pltpu.make_async_copy(table.at[idxs[k].at[s]], gbs[k].at[s],
                                      gsems[s]).wait()

            def grp(g, carry):
                slg = pl.ds(g * 16, 16)
                wav = wbuf[s, 0, slg]
                wbv = wbuf[s, 1, slg]
                wcv = wbuf[s, 2, slg]
                wdv = wbuf[s, 3, slg]
                wsv = wav + wbv + wcv + wdv
                gmax = jnp.max(wsv)
                pbase = g * 16

                @pl.when(gmax != 0.0)
                def _():
                    for l in range(16):
                        p = pbase + l
                        wa = wav[l]
                        wb = wbv[l]
                        wc = wcv[l]
                        wd = wdv[l]
                        ws = wsv[l]

                        @pl.when(ws != 0.0)
                        def _():
                            for v in range(C // 16):
                                slv = pl.ds(v * 16, 16)
                                acc = (wa * gb_a[s, p, slv]
                                       + wb * gb_b[s, p, slv]
                                       + wc * gb_c[s, p, slv]
                                       + wd * gb_d[s, p, slv])
                                obuf[s, p, slv] = acc

                        @pl.when(ws == 0.0)
                        def _():
                            for v in range(C // 16):
                                obuf[s, p, pl.ds(v * 16, 16)] = zv

                @pl.when(gmax == 0.0)
                def _():
                    for l in range(16):
                        p = pbase + l
                        for v in range(C // 16):
                            obuf[s, p, pl.ds(v * 16, 16)] = zv

                return carry

            lax.fori_loop(0, CHUNK // 16, grp, 0)
            pltpu.async_copy(obuf.at[s], out.at[pl.ds(pix_base, CHUNK)],
                             osems[s])

        @pl.when(fl == 0)
        def _():
            pltpu.async_copy(zbuf, out.at[pl.ds(pix_base, CHUNK)], osems[s])

    def pair_body(t, carry):
        c0 = t * 2
        c1 = t * 2 + 1
        coords_and_fire(c0, 0)
        coords_and_fire(c1, 1)
        drain_combine_out(c0, 0, t == 0)
        drain_combine_out(c1, 1, t == 0)
        return carry

    lax.fori_loop(0, CHUNKS_PER_TILE // 2, pair_body, 0)

    for s in range(2):
        pltpu.make_async_copy(obuf.at[s], out.at[pl.ds(tile_pix_base, CHUNK)],
                              osems[s]).wait()


@functools.partial(
    pl.kernel,
    out_type=jax.ShapeDtypeStruct((NPIX, C), jnp.float32),
    mesh=plsc.VectorSubcoreMesh(core_axis_name="c", subcore_axis_name="s",
                                num_cores=NC, num_subcores=NS),
    compiler_params=pltpu.CompilerParams(needs_layout_passes=False,
                                         use_tc_tiling_on_sc=True),
    scratch_types=[
        pltpu.VMEM((2, CHUNK), jnp.int32),
        pltpu.VMEM((2, CHUNK), jnp.int32),
        pltpu.VMEM((2, CHUNK), jnp.int32),
        pltpu.VMEM((2, CHUNK), jnp.int32),
        pltpu.VMEM((2, CHUNK, CP), jnp.float32),
        pltpu.VMEM((2, CHUNK, CP), jnp.float32),
        pltpu.VMEM((2, CHUNK, CP), jnp.float32),
        pltpu.VMEM((2, CHUNK, CP), jnp.float32),
        pltpu.VMEM((2, 4, CHUNK), jnp.float32),
        pltpu.VMEM((2, CHUNK, C), jnp.float32),
        pltpu.VMEM((CHUNK, C), jnp.float32),
        pltpu.VMEM((1, 16), jnp.float32),
        pltpu.SMEM((2,), jnp.int32),
        pltpu.SemaphoreType.DMA,
        pltpu.SemaphoreType.DMA,
        pltpu.SemaphoreType.DMA,
        pltpu.SemaphoreType.DMA,
    ],
)
def _stn_call(table, thetap, out, *scratch):
    _stn_body(table, thetap, out, *scratch)


def kernel(conv_input, theta):
    table = conv_input.reshape(NPIX, C)
    table = jnp.concatenate(
        [table, jnp.zeros((NPIX, CP - C), jnp.float32)], axis=1)
    thetap = jnp.zeros((B, 16), jnp.float32).at[:, :6].set(
        theta.astype(jnp.float32))
    out = _stn_call(table, thetap)
    return out.reshape(B, H, W, C)
